# baseline (speedup 1.0000x reference)
import functools

import jax
import jax.numpy as jnp
from jax import lax
from jax.experimental import pallas as pl
from jax.experimental.pallas import tpu as pltpu

_ROWS = 65536
_D = 256
_BUFROWS = 32728
_CHS = [(0, _BUFROWS), (_BUFROWS, _BUFROWS), (2 * _BUFROWS, _ROWS - 2 * _BUFROWS)]
_NBUF = 2


def _ring_body(y_ref, c_ref, w_ref, out_ref, buf0, buf1, yv,
               in_sems, out_sems, ysem):
    bufs = [buf0, buf1]

    def in_copy(k):
        off, sz = _CHS[k]
        return pltpu.make_async_copy(
            w_ref.at[pl.ds(off, sz)], bufs[k % _NBUF].at[pl.ds(0, sz)],
            in_sems.at[k % _NBUF])

    def out_copy(k):
        off, sz = _CHS[k]
        return pltpu.make_async_copy(
            bufs[k % _NBUF].at[pl.ds(0, sz)], out_ref.at[pl.ds(off, sz)],
            out_sems.at[k % _NBUF])

    ycp = pltpu.make_async_copy(y_ref.at[pl.ds(1, 3)], yv, ysem)
    ycp.start()
    for k in range(_NBUF):
        in_copy(k).start()
    ycp.wait()

    for k in range(len(_CHS)):
        in_copy(k).wait()
        if k == 0:
            c17 = c_ref[17]
            c18 = c_ref[18]
            c19 = c_ref[19]
            buf0[0:1, :] = jnp.full((1, _D), c19 * c17, jnp.float32)
            buf0[1:2, :] = jnp.full((1, _D), c18 / c19, jnp.float32)
            buf0[2:3, :] = yv[2:3, :] + yv[0:1, :] + 2.0 * yv[1:2, :]
        out_copy(k).start()
        if k + _NBUF < len(_CHS):
            out_copy(k).wait()
            in_copy(k + _NBUF).start()
    for k in range(max(0, len(_CHS) - _NBUF), len(_CHS)):
        out_copy(k).wait()


def kernel(y, w, c, t):
    del t
    return pl.pallas_call(
        _ring_body,
        out_shape=jax.ShapeDtypeStruct((_ROWS, _D), jnp.float32),
        in_specs=[
            pl.BlockSpec(memory_space=pl.ANY),
            pl.BlockSpec(memory_space=pltpu.SMEM),
            pl.BlockSpec(memory_space=pl.ANY),
        ],
        out_specs=pl.BlockSpec(memory_space=pl.ANY),
        scratch_shapes=[
            pltpu.VMEM((_BUFROWS, _D), jnp.float32),
            pltpu.VMEM((_BUFROWS, _D), jnp.float32),
            pltpu.VMEM((3, _D), jnp.float32),
            pltpu.SemaphoreType.DMA((_NBUF,)),
            pltpu.SemaphoreType.DMA((_NBUF,)),
            pltpu.SemaphoreType.DMA,
        ],
        compiler_params=pltpu.CompilerParams(
            vmem_limit_bytes=134217728,
        ),
    )(y, c, w)


# speedup vs baseline: 1.0421x; 1.0421x over previous
import functools

import jax
import jax.numpy as jnp
from jax import lax
from jax.experimental import pallas as pl
from jax.experimental.pallas import tpu as pltpu

_ROWS = 65536
_D = 256
_SIZES = [2048, 4096, 8192, 14336, 14336, 14336, 6144, 2048]
_BUFROWS = max(_SIZES)
_CHS = []
_off = 0
for _s in _SIZES:
    _CHS.append((_off, _s))
    _off += _s
assert _off == _ROWS
_NBUF = 3


def _ring_body(y_ref, c_ref, w_ref, out_ref, buf0, buf1, buf2, yv,
               in_sems, out_sems, ysem):
    bufs = [buf0, buf1, buf2]

    def in_copy(k):
        off, sz = _CHS[k]
        return pltpu.make_async_copy(
            w_ref.at[pl.ds(off, sz)], bufs[k % _NBUF].at[pl.ds(0, sz)],
            in_sems.at[k % _NBUF])

    def out_copy(k):
        off, sz = _CHS[k]
        return pltpu.make_async_copy(
            bufs[k % _NBUF].at[pl.ds(0, sz)], out_ref.at[pl.ds(off, sz)],
            out_sems.at[k % _NBUF])

    ycp = pltpu.make_async_copy(y_ref.at[pl.ds(1, 3)], yv, ysem)
    ycp.start()
    for k in range(_NBUF):
        in_copy(k).start()
    ycp.wait()

    for k in range(len(_CHS)):
        in_copy(k).wait()
        if k == 0:
            c17 = c_ref[17]
            c18 = c_ref[18]
            c19 = c_ref[19]
            buf0[0:1, :] = jnp.full((1, _D), c19 * c17, jnp.float32)
            buf0[1:2, :] = jnp.full((1, _D), c18 / c19, jnp.float32)
            buf0[2:3, :] = yv[2:3, :] + yv[0:1, :] + 2.0 * yv[1:2, :]
        out_copy(k).start()
        if k + _NBUF < len(_CHS):
            out_copy(k).wait()
            in_copy(k + _NBUF).start()
    for k in range(max(0, len(_CHS) - _NBUF), len(_CHS)):
        out_copy(k).wait()


def kernel(y, w, c, t):
    del t
    return pl.pallas_call(
        _ring_body,
        out_shape=jax.ShapeDtypeStruct((_ROWS, _D), jnp.float32),
        in_specs=[
            pl.BlockSpec(memory_space=pl.ANY),
            pl.BlockSpec(memory_space=pltpu.SMEM),
            pl.BlockSpec(memory_space=pl.ANY),
        ],
        out_specs=pl.BlockSpec(memory_space=pl.ANY),
        scratch_shapes=[
            pltpu.VMEM((_BUFROWS, _D), jnp.float32),
            pltpu.VMEM((_BUFROWS, _D), jnp.float32),
            pltpu.VMEM((_BUFROWS, _D), jnp.float32),
            pltpu.VMEM((3, _D), jnp.float32),
            pltpu.SemaphoreType.DMA((_NBUF,)),
            pltpu.SemaphoreType.DMA((_NBUF,)),
            pltpu.SemaphoreType.DMA,
        ],
        compiler_params=pltpu.CompilerParams(
            vmem_limit_bytes=134217728,
        ),
    )(y, c, w)


# speedup vs baseline: 1.0604x; 1.0176x over previous
import functools

import jax
import jax.numpy as jnp
from jax import lax
from jax.experimental import pallas as pl
from jax.experimental.pallas import tpu as pltpu

_ROWS = 65536
_D = 256
_SIZES = [2048, 16384, 16384, 16384, 14336]
_BUFROWS = max(_SIZES)
_CHS = []
_off = 0
for _s in _SIZES:
    _CHS.append((_off, _s))
    _off += _s
assert _off == _ROWS
_NBUF = 3


def _ring_body(y_ref, c_ref, w_ref, out_ref, buf0, buf1, buf2, yv,
               in_sems, out_sems, ysem):
    bufs = [buf0, buf1, buf2]

    def in_copy(k):
        off, sz = _CHS[k]
        return pltpu.make_async_copy(
            w_ref.at[pl.ds(off, sz)], bufs[k % _NBUF].at[pl.ds(0, sz)],
            in_sems.at[k % _NBUF])

    def out_copy(k):
        off, sz = _CHS[k]
        return pltpu.make_async_copy(
            bufs[k % _NBUF].at[pl.ds(0, sz)], out_ref.at[pl.ds(off, sz)],
            out_sems.at[k % _NBUF])

    ycp = pltpu.make_async_copy(y_ref.at[pl.ds(1, 3)], yv, ysem)
    ycp.start()
    for k in range(_NBUF):
        in_copy(k).start()
    ycp.wait()

    for k in range(len(_CHS)):
        in_copy(k).wait()
        if k == 0:
            c17 = c_ref[17]
            c18 = c_ref[18]
            c19 = c_ref[19]
            buf0[0:1, :] = jnp.full((1, _D), c19 * c17, jnp.float32)
            buf0[1:2, :] = jnp.full((1, _D), c18 / c19, jnp.float32)
            buf0[2:3, :] = yv[2:3, :] + yv[0:1, :] + 2.0 * yv[1:2, :]
        out_copy(k).start()
        if k + _NBUF < len(_CHS):
            out_copy(k).wait()
            in_copy(k + _NBUF).start()
    for k in range(max(0, len(_CHS) - _NBUF), len(_CHS)):
        out_copy(k).wait()


def kernel(y, w, c, t):
    del t
    return pl.pallas_call(
        _ring_body,
        out_shape=jax.ShapeDtypeStruct((_ROWS, _D), jnp.float32),
        in_specs=[
            pl.BlockSpec(memory_space=pl.ANY),
            pl.BlockSpec(memory_space=pltpu.SMEM),
            pl.BlockSpec(memory_space=pl.ANY),
        ],
        out_specs=pl.BlockSpec(memory_space=pl.ANY),
        scratch_shapes=[
            pltpu.VMEM((_BUFROWS, _D), jnp.float32),
            pltpu.VMEM((_BUFROWS, _D), jnp.float32),
            pltpu.VMEM((_BUFROWS, _D), jnp.float32),
            pltpu.VMEM((3, _D), jnp.float32),
            pltpu.SemaphoreType.DMA((_NBUF,)),
            pltpu.SemaphoreType.DMA((_NBUF,)),
            pltpu.SemaphoreType.DMA,
        ],
        compiler_params=pltpu.CompilerParams(
            vmem_limit_bytes=134217728,
        ),
    )(y, c, w)


# speedup vs baseline: 1.0641x; 1.0035x over previous
import functools

import jax
import jax.numpy as jnp
from jax import lax
from jax.experimental import pallas as pl
from jax.experimental.pallas import tpu as pltpu

_ROWS = 65536
_D = 256
_SIZES = [16384, 16384, 16384, 16384]
_BUFROWS = max(_SIZES)
_CHS = []
_off = 0
for _s in _SIZES:
    _CHS.append((_off, _s))
    _off += _s
assert _off == _ROWS
_NBUF = 3


def _ring_body(y_ref, c_ref, w_ref, out_ref, buf0, buf1, buf2, yv,
               in_sems, out_sems, ysem):
    bufs = [buf0, buf1, buf2]

    def in_copy(k):
        off, sz = _CHS[k]
        return pltpu.make_async_copy(
            w_ref.at[pl.ds(off, sz)], bufs[k % _NBUF].at[pl.ds(0, sz)],
            in_sems.at[k % _NBUF])

    def out_copy(k):
        off, sz = _CHS[k]
        return pltpu.make_async_copy(
            bufs[k % _NBUF].at[pl.ds(0, sz)], out_ref.at[pl.ds(off, sz)],
            out_sems.at[k % _NBUF])

    ycp = pltpu.make_async_copy(y_ref.at[pl.ds(1, 3)], yv, ysem)
    ycp.start()
    for k in range(_NBUF):
        in_copy(k).start()
    ycp.wait()

    for k in range(len(_CHS)):
        in_copy(k).wait()
        if k == 0:
            c17 = c_ref[17]
            c18 = c_ref[18]
            c19 = c_ref[19]
            buf0[0:1, :] = jnp.full((1, _D), c19 * c17, jnp.float32)
            buf0[1:2, :] = jnp.full((1, _D), c18 / c19, jnp.float32)
            buf0[2:3, :] = yv[2:3, :] + yv[0:1, :] + 2.0 * yv[1:2, :]
        out_copy(k).start()
        if k + _NBUF < len(_CHS):
            out_copy(k).wait()
            in_copy(k + _NBUF).start()
    for k in range(max(0, len(_CHS) - _NBUF), len(_CHS)):
        out_copy(k).wait()


def kernel(y, w, c, t):
    del t
    return pl.pallas_call(
        _ring_body,
        out_shape=jax.ShapeDtypeStruct((_ROWS, _D), jnp.float32),
        in_specs=[
            pl.BlockSpec(memory_space=pl.ANY),
            pl.BlockSpec(memory_space=pltpu.SMEM),
            pl.BlockSpec(memory_space=pl.ANY),
        ],
        out_specs=pl.BlockSpec(memory_space=pl.ANY),
        scratch_shapes=[
            pltpu.VMEM((_BUFROWS, _D), jnp.float32),
            pltpu.VMEM((_BUFROWS, _D), jnp.float32),
            pltpu.VMEM((_BUFROWS, _D), jnp.float32),
            pltpu.VMEM((3, _D), jnp.float32),
            pltpu.SemaphoreType.DMA((_NBUF,)),
            pltpu.SemaphoreType.DMA((_NBUF,)),
            pltpu.SemaphoreType.DMA,
        ],
        compiler_params=pltpu.CompilerParams(
            vmem_limit_bytes=134217728,
        ),
    )(y, c, w)
